# baseline (speedup 1.0000x reference)
import functools

import jax
import jax.numpy as jnp
from jax import lax
from jax.experimental import pallas as pl
from jax.experimental.pallas import tpu as pltpu
from jax.experimental.pallas import tpu_sc as plsc

NC = 2
NS = 16
NW = NC * NS
C = 40
RAW = 128


HALF = RAW // 2


def _sc_gather(table, row, col):
    E = row.shape[0]
    per_w = E // NW
    n_pairs = per_w // (2 * C)
    mesh = plsc.VectorSubcoreMesh(
        core_axis_name="c", subcore_axis_name="s", num_cores=NC,
        num_subcores=NS)
    fdt = jax.ShapeDtypeStruct((E, RAW), jnp.float32)

    @functools.partial(
        pl.kernel, mesh=mesh,
        out_type=(fdt, fdt),
        scratch_types=[
            pltpu.VMEM((C,), jnp.int32),
            pltpu.VMEM((C,), jnp.int32),
            pltpu.VMEM((C,), jnp.int32),
            pltpu.VMEM((C,), jnp.int32),
            pltpu.VMEM((C, RAW), jnp.float32),
            pltpu.VMEM((C, RAW), jnp.float32),
            pltpu.VMEM((C, RAW), jnp.float32),
            pltpu.VMEM((C, RAW), jnp.float32),
            pltpu.SemaphoreType.DMA,
            pltpu.SemaphoreType.DMA,
        ],
    )
    def k(t_hbm, row_hbm, col_hbm, gr_out, gc_out,
          ir_a, ic_a, ir_b, ic_b, gr_a, gc_a, gr_b, gc_b, sem_a, sem_b):
        wid = lax.axis_index("s") * NC + lax.axis_index("c")
        w_base = wid * per_w

        def drain_b():
            pltpu.make_async_copy(t_hbm.at[pl.ds(0, C)], gr_b, sem_b).wait()
            pltpu.make_async_copy(t_hbm.at[pl.ds(0, C)], gc_b, sem_b).wait()

        def body(i, _):
            a = w_base + (2 * i) * C
            b = a + C
            pltpu.sync_copy(row_hbm.at[pl.ds(a, C)], ir_a)
            pltpu.sync_copy(col_hbm.at[pl.ds(a, C)], ic_a)
            da0 = pltpu.async_copy(t_hbm.at[ir_a], gr_a, sem_a)
            da1 = pltpu.async_copy(t_hbm.at[ic_a], gc_a, sem_a)

            @pl.when(i > 0)
            def _prev():
                bp = a - C
                drain_b()
                pltpu.sync_copy(gr_b, gr_out.at[pl.ds(bp, C)])
                pltpu.sync_copy(gc_b, gc_out.at[pl.ds(bp, C)])

            pltpu.sync_copy(row_hbm.at[pl.ds(b, C)], ir_b)
            pltpu.sync_copy(col_hbm.at[pl.ds(b, C)], ic_b)
            pltpu.async_copy(t_hbm.at[ir_b], gr_b, sem_b)
            pltpu.async_copy(t_hbm.at[ic_b], gc_b, sem_b)
            da0.wait()
            da1.wait()
            pltpu.sync_copy(gr_a, gr_out.at[pl.ds(a, C)])
            pltpu.sync_copy(gc_a, gc_out.at[pl.ds(a, C)])
            return _

        lax.fori_loop(0, n_pairs, body, None)
        bl = w_base + per_w - C
        drain_b()
        pltpu.sync_copy(gr_b, gr_out.at[pl.ds(bl, C)])
        pltpu.sync_copy(gc_b, gc_out.at[pl.ds(bl, C)])

    return k(table, row, col)


def _tc_mlp(grow, gcol, row3, n_proto, trash,
            w1a, w1b, b1, w2, b2, tw1, tb1, tw2, tb2, tw3):
    E = grow.shape[0]
    B = 2000
    grid = (E // B,)

    def unpack(packed):
        u = lax.bitcast_convert_type(packed, jnp.int32)
        lo = lax.bitcast_convert_type(u << 16, jnp.float32)
        hi = lax.bitcast_convert_type(u & jnp.int32(-65536), jnp.float32)
        return jnp.concatenate([lo, hi], axis=1)

    def body(gr_ref, gc_ref, row_ref, w1a_ref, w1b_ref, b1_ref,
             w2_ref, b2_ref, tw1_ref, tb1_ref, tw2_ref, tb2_ref, tw3_ref,
             out_ref, rowq_ref):
        r = row_ref[0, 0, :]
        rowq_ref[0, 0, :] = jnp.where(r >= n_proto, r - n_proto, trash)
        diff = unpack(gr_ref[:, :HALF]) - unpack(gc_ref[:, :HALF])
        sqd = jnp.sum(diff * diff, axis=1, keepdims=True)
        h = jnp.dot(unpack(gc_ref[:, HALF:]).astype(jnp.bfloat16),
                    w1a_ref[...], preferred_element_type=jnp.float32)
        h = h + sqd * w1b_ref[...] + b1_ref[...]
        h = h * jax.nn.sigmoid(h)
        h = jnp.dot(h.astype(jnp.bfloat16), w2_ref[...],
                    preferred_element_type=jnp.float32)
        h = h + b2_ref[...]
        h = h * jax.nn.sigmoid(h)
        h = jnp.dot(h.astype(jnp.bfloat16), tw1_ref[...],
                    preferred_element_type=jnp.float32)
        h = h + tb1_ref[...]
        h = h * jax.nn.sigmoid(h)
        h = jnp.dot(h.astype(jnp.bfloat16), tw2_ref[...],
                    preferred_element_type=jnp.float32)
        h = h + tb2_ref[...]
        h = h * jax.nn.sigmoid(h)
        t = jnp.sum(h * tw3_ref[...], axis=1, keepdims=True)
        out_ref[...] = diff * t

    blk_o = pl.BlockSpec((B, RAW), lambda i: (i, 0))
    blk_r = pl.BlockSpec((1, 1, B), lambda i: (i, 0, 0))
    full = lambda shape: pl.BlockSpec(shape, lambda i: tuple(0 for _ in shape))
    return pl.pallas_call(
        body,
        grid=grid,
        in_specs=[
            blk_o, blk_o, blk_r,
            full((RAW, 64)), full((1, 64)), full((1, 64)),
            full((64, 64)), full((1, 64)),
            full((64, 64)), full((1, 64)),
            full((64, 64)), full((1, 64)),
            full((1, 64)),
        ],
        out_specs=(blk_o, blk_r),
        out_shape=(jax.ShapeDtypeStruct((E, RAW), jnp.float32),
                   jax.ShapeDtypeStruct((E // B, 1, B), jnp.int32)),
    )(grow, gcol, row3, w1a, w1b, b1, w2, b2, tw1, tb1, tw2, tb2, tw3)


def _sc_scatter(trans, rowq, zrow, ones_c, NQ):
    E, _ = trans.shape
    per_w = E // NW
    n_chunks = per_w // C
    rows_per_tile = NQ // NS
    n_init = rows_per_tile // C
    mesh = plsc.VectorSubcoreMesh(
        core_axis_name="c", subcore_axis_name="s", num_cores=NC,
        num_subcores=NS)
    fdt = jax.ShapeDtypeStruct((NC, NQ, RAW), jnp.float32)

    n_pairs = n_chunks // 2

    @functools.partial(
        pl.kernel, mesh=mesh,
        out_type=(fdt, fdt),
        scratch_types=[
            pltpu.VMEM_SHARED((NQ, RAW), jnp.float32),
            pltpu.VMEM_SHARED((NQ, RAW), jnp.float32),
            pltpu.VMEM((C,), jnp.int32),
            pltpu.VMEM((C,), jnp.int32),
            pltpu.VMEM((C, RAW), jnp.float32),
            pltpu.VMEM((C, RAW), jnp.float32),
            pltpu.VMEM((C, RAW), jnp.float32),
            pltpu.SemaphoreType.DMA,
            pltpu.SemaphoreType.DMA,
        ],
    )
    def k(trans_hbm, rowq_hbm, zrow_hbm, ones_hbm, s_out, cnt_out,
          acc_sh, cnt_sh, idx_a, idx_b, tr_a, tr_b, ones_v, sem_a, sem_b):
        cid = lax.axis_index("c")
        sid = lax.axis_index("s")
        wid = sid * NC + cid
        w_base = wid * per_w
        tile_rows = sid * rows_per_tile

        pltpu.sync_copy(zrow_hbm, tr_a)

        def zbody(j, _):
            pltpu.sync_copy(tr_a, acc_sh.at[pl.ds(tile_rows + j * C, C)])
            pltpu.sync_copy(tr_a, cnt_sh.at[pl.ds(tile_rows + j * C, C)])
            return _

        lax.fori_loop(0, n_init, zbody, None)
        pltpu.sync_copy(ones_hbm, ones_v)
        plsc.subcore_barrier()

        def scat_b():
            pltpu.make_async_copy(
                trans_hbm.at[pl.ds(0, C)], tr_b, sem_b).wait()
            pltpu.sync_copy(tr_b, acc_sh.at[idx_b], add=True)
            pltpu.sync_copy(ones_v, cnt_sh.at[idx_b], add=True)

        def body(i, _):
            a = w_base + (2 * i) * C
            b = a + C
            pltpu.sync_copy(rowq_hbm.at[pl.ds(a, C)], idx_a)
            da = pltpu.async_copy(trans_hbm.at[pl.ds(a, C)], tr_a, sem_a)

            @pl.when(i > 0)
            def _prev():
                scat_b()

            pltpu.sync_copy(rowq_hbm.at[pl.ds(b, C)], idx_b)
            pltpu.async_copy(trans_hbm.at[pl.ds(b, C)], tr_b, sem_b)
            da.wait()
            pltpu.sync_copy(tr_a, acc_sh.at[idx_a], add=True)
            pltpu.sync_copy(ones_v, cnt_sh.at[idx_a], add=True)
            return _

        lax.fori_loop(0, n_pairs, body, None)
        scat_b()
        plsc.subcore_barrier()

        def obody(j, _):
            r = tile_rows + j * C
            pltpu.sync_copy(acc_sh.at[pl.ds(r, C)], tr_a)
            pltpu.sync_copy(tr_a, s_out.at[cid, pl.ds(r, C)])
            pltpu.sync_copy(cnt_sh.at[pl.ds(r, C)], ones_v)
            pltpu.sync_copy(ones_v, cnt_out.at[cid, pl.ds(r, C)])
            return _

        lax.fori_loop(0, n_init, obody, None)

    return k(trans, rowq, zrow, ones_c)


def _tc_finish(qry, s_parts, cnt_parts):
    nq = qry.shape[0]

    def body(qry_ref, s_ref, cnt_ref, out_ref):
        s = s_ref[0] + s_ref[1]
        cnt = cnt_ref[0, :, 0:1] + cnt_ref[1, :, 0:1]
        out_ref[...] = qry_ref[...] + s / jnp.maximum(cnt, 1.0)

    return pl.pallas_call(
        body,
        grid=(1,),
        in_specs=[
            pl.BlockSpec((nq, RAW), lambda i: (0, 0)),
            pl.BlockSpec((NC, nq, RAW), lambda i: (0, 0, 0)),
            pl.BlockSpec((NC, nq, RAW), lambda i: (0, 0, 0)),
        ],
        out_specs=pl.BlockSpec((nq, RAW), lambda i: (0, 0)),
        out_shape=jax.ShapeDtypeStruct((nq, RAW), jnp.float32),
    )(qry, s_parts, cnt_parts)


def kernel(edge_index, neighbor, qry_embeds, proto_embeds,
           msg_W1, msg_b1, msg_W2, msg_b2,
           trans_W1, trans_b1, trans_W2, trans_b2, trans_W3):
    n_proto = proto_embeds.shape[0]
    x = jnp.concatenate([proto_embeds, qry_embeds], axis=0)
    xn = jnp.concatenate([proto_embeds, neighbor], axis=0)
    E = edge_index.shape[1]

    def pack(a):
        ab = a.astype(jnp.bfloat16)
        pair = jnp.stack([ab[:, :HALF], ab[:, HALF:]], axis=-1)
        return lax.bitcast_convert_type(pair, jnp.float32)

    table = jnp.concatenate([pack(x), pack(xn)], axis=1)
    grow, gcol = _sc_gather(table, edge_index[0], edge_index[1])

    nq = qry_embeds.shape[0]
    nq_pad = ((nq + 1 + NS * C - 1) // (NS * C)) * (NS * C)
    B = 2000
    row3 = edge_index[0].reshape(E // B, 1, B)
    bf = jnp.bfloat16
    w1a = msg_W1[:RAW].astype(bf)
    w1b = msg_W1[RAW:RAW + 1]
    trans, rowq3 = _tc_mlp(grow, gcol, row3, n_proto, nq_pad - 1,
                           w1a, w1b, msg_b1.reshape(1, -1),
                           msg_W2.astype(bf), msg_b2.reshape(1, -1),
                           trans_W1.astype(bf), trans_b1.reshape(1, -1),
                           trans_W2.astype(bf), trans_b2.reshape(1, -1),
                           trans_W3.reshape(1, -1))

    zrow = jnp.zeros((C, RAW), jnp.float32)
    ones_c = jnp.ones((C, RAW), jnp.float32)
    s_parts, cnt_parts = _sc_scatter(trans, rowq3.reshape(E), zrow, ones_c,
                                     nq_pad)

    qry_new = _tc_finish(qry_embeds, s_parts, cnt_parts)
    return (neighbor, qry_new)


# speedup vs baseline: 1.4199x; 1.4199x over previous
import functools

import jax
import jax.numpy as jnp
from jax import lax
from jax.experimental import pallas as pl
from jax.experimental.pallas import tpu as pltpu
from jax.experimental.pallas import tpu_sc as plsc

NC = 2
NS = 16
NW = NC * NS
C = 40
RAW = 128


HALF = RAW // 2


def _sc_gather(table, row, col):
    E = row.shape[0]
    per_w = E // NW
    n_pairs = per_w // (2 * C)
    mesh = plsc.VectorSubcoreMesh(
        core_axis_name="c", subcore_axis_name="s", num_cores=NC,
        num_subcores=NS)
    fdt = jax.ShapeDtypeStruct((E, RAW), jnp.float32)

    @functools.partial(
        pl.kernel, mesh=mesh,
        out_type=(fdt, fdt),
        scratch_types=[
            pltpu.VMEM((C,), jnp.int32),
            pltpu.VMEM((C,), jnp.int32),
            pltpu.VMEM((C,), jnp.int32),
            pltpu.VMEM((C,), jnp.int32),
            pltpu.VMEM((C, RAW), jnp.float32),
            pltpu.VMEM((C, RAW), jnp.float32),
            pltpu.VMEM((C, RAW), jnp.float32),
            pltpu.VMEM((C, RAW), jnp.float32),
            pltpu.SemaphoreType.DMA,
            pltpu.SemaphoreType.DMA,
        ],
    )
    def k(t_hbm, row_hbm, col_hbm, gr_out, gc_out,
          ir_a, ic_a, ir_b, ic_b, gr_a, gc_a, gr_b, gc_b, sem_a, sem_b):
        wid = lax.axis_index("s") * NC + lax.axis_index("c")
        w_base = wid * per_w

        def drain_b():
            pltpu.make_async_copy(t_hbm.at[pl.ds(0, C)], gr_b, sem_b).wait()
            pltpu.make_async_copy(t_hbm.at[pl.ds(0, C)], gc_b, sem_b).wait()

        def body(i, _):
            a = w_base + (2 * i) * C
            b = a + C
            pltpu.sync_copy(row_hbm.at[pl.ds(a, C)], ir_a)
            pltpu.sync_copy(col_hbm.at[pl.ds(a, C)], ic_a)
            da0 = pltpu.async_copy(t_hbm.at[ir_a], gr_a, sem_a)
            da1 = pltpu.async_copy(t_hbm.at[ic_a], gc_a, sem_a)

            @pl.when(i > 0)
            def _prev():
                bp = a - C
                drain_b()
                pltpu.sync_copy(gr_b, gr_out.at[pl.ds(bp, C)])
                pltpu.sync_copy(gc_b, gc_out.at[pl.ds(bp, C)])

            pltpu.sync_copy(row_hbm.at[pl.ds(b, C)], ir_b)
            pltpu.sync_copy(col_hbm.at[pl.ds(b, C)], ic_b)
            pltpu.async_copy(t_hbm.at[ir_b], gr_b, sem_b)
            pltpu.async_copy(t_hbm.at[ic_b], gc_b, sem_b)
            da0.wait()
            da1.wait()
            pltpu.sync_copy(gr_a, gr_out.at[pl.ds(a, C)])
            pltpu.sync_copy(gc_a, gc_out.at[pl.ds(a, C)])
            return _

        lax.fori_loop(0, n_pairs, body, None)
        bl = w_base + per_w - C
        drain_b()
        pltpu.sync_copy(gr_b, gr_out.at[pl.ds(bl, C)])
        pltpu.sync_copy(gc_b, gc_out.at[pl.ds(bl, C)])

    return k(table, row, col)


def _tc_pre(xn, w1a):
    n = xn.shape[0]
    BP = 2000

    def body(xn_ref, w_ref, out_ref):
        out_ref[...] = jnp.dot(xn_ref[...], w_ref[...],
                               preferred_element_type=jnp.float32)

    return pl.pallas_call(
        body,
        grid=(n // BP,),
        in_specs=[pl.BlockSpec((BP, RAW), lambda i: (i, 0)),
                  pl.BlockSpec((RAW, HALF), lambda i: (0, 0))],
        out_specs=pl.BlockSpec((BP, HALF), lambda i: (i, 0)),
        out_shape=jax.ShapeDtypeStruct((n, HALF), jnp.float32),
    )(xn, w1a)


def _tc_mlp(grow, gcol, row3, n_proto, trash,
            w1b, b1, w2, b2, tw1, tb1, tw2, tb2, tw3):
    E = grow.shape[0]
    B = 4000
    grid = (E // B,)

    def unpack(packed):
        u = lax.bitcast_convert_type(packed, jnp.int32)
        lo = lax.bitcast_convert_type(u << 16, jnp.float32)
        hi = lax.bitcast_convert_type(u & jnp.int32(-65536), jnp.float32)
        return jnp.concatenate([lo, hi], axis=1)

    def body(gr_ref, gc_ref, row_ref, w1b_ref, b1_ref,
             w2_ref, b2_ref, tw1_ref, tb1_ref, tw2_ref, tb2_ref, tw3_ref,
             out_ref, rowq_ref):
        r = row_ref[0, 0, :]
        rowq_ref[0, 0, :] = jnp.where(r >= n_proto, r - n_proto, trash)
        diff = unpack(gr_ref[:, :HALF]) - unpack(gc_ref[:, :HALF])
        sqd = jnp.sum(diff * diff, axis=1, keepdims=True)
        h = gc_ref[:, HALF:] + sqd * w1b_ref[...] + b1_ref[...]
        h = h * jax.nn.sigmoid(h)
        h = jnp.dot(h.astype(jnp.bfloat16), w2_ref[...],
                    preferred_element_type=jnp.float32)
        h = h + b2_ref[...]
        h = h * jax.nn.sigmoid(h)
        h = jnp.dot(h.astype(jnp.bfloat16), tw1_ref[...],
                    preferred_element_type=jnp.float32)
        h = h + tb1_ref[...]
        h = h * jax.nn.sigmoid(h)
        h = jnp.dot(h.astype(jnp.bfloat16), tw2_ref[...],
                    preferred_element_type=jnp.float32)
        h = h + tb2_ref[...]
        h = h * jax.nn.sigmoid(h)
        t = jnp.sum(h * tw3_ref[...], axis=1, keepdims=True)
        out_ref[...] = diff * t

    blk_o = pl.BlockSpec((B, RAW), lambda i: (i, 0))
    blk_r = pl.BlockSpec((1, 1, B), lambda i: (i, 0, 0))
    full = lambda shape: pl.BlockSpec(shape, lambda i: tuple(0 for _ in shape))
    return pl.pallas_call(
        body,
        grid=grid,
        in_specs=[
            blk_o, blk_o, blk_r,
            full((1, 64)), full((1, 64)),
            full((64, 64)), full((1, 64)),
            full((64, 64)), full((1, 64)),
            full((64, 64)), full((1, 64)),
            full((1, 64)),
        ],
        out_specs=(blk_o, blk_r),
        out_shape=(jax.ShapeDtypeStruct((E, RAW), jnp.float32),
                   jax.ShapeDtypeStruct((E // B, 1, B), jnp.int32)),
    )(grow, gcol, row3, w1b, b1, w2, b2, tw1, tb1, tw2, tb2, tw3)


def _sc_scatter(trans, rowq, zrow, ones_c, NQ):
    E, _ = trans.shape
    per_w = E // NW
    n_chunks = per_w // C
    rows_per_tile = NQ // NS
    n_init = rows_per_tile // C
    mesh = plsc.VectorSubcoreMesh(
        core_axis_name="c", subcore_axis_name="s", num_cores=NC,
        num_subcores=NS)
    fdt = jax.ShapeDtypeStruct((NC, NQ, RAW), jnp.float32)

    n_pairs = n_chunks // 2

    @functools.partial(
        pl.kernel, mesh=mesh,
        out_type=(fdt, fdt),
        scratch_types=[
            pltpu.VMEM_SHARED((NQ, RAW), jnp.float32),
            pltpu.VMEM_SHARED((NQ, RAW), jnp.float32),
            pltpu.VMEM((C,), jnp.int32),
            pltpu.VMEM((C,), jnp.int32),
            pltpu.VMEM((C, RAW), jnp.float32),
            pltpu.VMEM((C, RAW), jnp.float32),
            pltpu.VMEM((C, RAW), jnp.float32),
            pltpu.SemaphoreType.DMA,
            pltpu.SemaphoreType.DMA,
        ],
    )
    def k(trans_hbm, rowq_hbm, zrow_hbm, ones_hbm, s_out, cnt_out,
          acc_sh, cnt_sh, idx_a, idx_b, tr_a, tr_b, ones_v, sem_a, sem_b):
        cid = lax.axis_index("c")
        sid = lax.axis_index("s")
        wid = sid * NC + cid
        w_base = wid * per_w
        tile_rows = sid * rows_per_tile

        pltpu.sync_copy(zrow_hbm, tr_a)

        def zbody(j, _):
            pltpu.sync_copy(tr_a, acc_sh.at[pl.ds(tile_rows + j * C, C)])
            pltpu.sync_copy(tr_a, cnt_sh.at[pl.ds(tile_rows + j * C, C)])
            return _

        lax.fori_loop(0, n_init, zbody, None)
        pltpu.sync_copy(ones_hbm, ones_v)
        plsc.subcore_barrier()

        def scat_b():
            pltpu.make_async_copy(
                trans_hbm.at[pl.ds(0, C)], tr_b, sem_b).wait()
            pltpu.sync_copy(tr_b, acc_sh.at[idx_b], add=True)
            pltpu.sync_copy(ones_v, cnt_sh.at[idx_b], add=True)

        def body(i, _):
            a = w_base + (2 * i) * C
            b = a + C
            pltpu.sync_copy(rowq_hbm.at[pl.ds(a, C)], idx_a)
            da = pltpu.async_copy(trans_hbm.at[pl.ds(a, C)], tr_a, sem_a)

            @pl.when(i > 0)
            def _prev():
                scat_b()

            pltpu.sync_copy(rowq_hbm.at[pl.ds(b, C)], idx_b)
            pltpu.async_copy(trans_hbm.at[pl.ds(b, C)], tr_b, sem_b)
            da.wait()
            pltpu.sync_copy(tr_a, acc_sh.at[idx_a], add=True)
            pltpu.sync_copy(ones_v, cnt_sh.at[idx_a], add=True)
            return _

        lax.fori_loop(0, n_pairs, body, None)
        scat_b()
        plsc.subcore_barrier()

        def obody(j, _):
            r = tile_rows + j * C
            pltpu.sync_copy(acc_sh.at[pl.ds(r, C)], tr_a)
            pltpu.sync_copy(tr_a, s_out.at[cid, pl.ds(r, C)])
            pltpu.sync_copy(cnt_sh.at[pl.ds(r, C)], ones_v)
            pltpu.sync_copy(ones_v, cnt_out.at[cid, pl.ds(r, C)])
            return _

        lax.fori_loop(0, n_init, obody, None)

    return k(trans, rowq, zrow, ones_c)


def _tc_finish(qry, s_parts, cnt_parts):
    nq = qry.shape[0]

    def body(qry_ref, s_ref, cnt_ref, out_ref):
        s = s_ref[0] + s_ref[1]
        cnt = cnt_ref[0, :, 0:1] + cnt_ref[1, :, 0:1]
        out_ref[...] = qry_ref[...] + s / jnp.maximum(cnt, 1.0)

    return pl.pallas_call(
        body,
        grid=(1,),
        in_specs=[
            pl.BlockSpec((nq, RAW), lambda i: (0, 0)),
            pl.BlockSpec((NC, nq, RAW), lambda i: (0, 0, 0)),
            pl.BlockSpec((NC, nq, RAW), lambda i: (0, 0, 0)),
        ],
        out_specs=pl.BlockSpec((nq, RAW), lambda i: (0, 0)),
        out_shape=jax.ShapeDtypeStruct((nq, RAW), jnp.float32),
    )(qry, s_parts, cnt_parts)


def kernel(edge_index, neighbor, qry_embeds, proto_embeds,
           msg_W1, msg_b1, msg_W2, msg_b2,
           trans_W1, trans_b1, trans_W2, trans_b2, trans_W3):
    n_proto = proto_embeds.shape[0]
    x = jnp.concatenate([proto_embeds, qry_embeds], axis=0)
    xn = jnp.concatenate([proto_embeds, neighbor], axis=0)
    E = edge_index.shape[1]

    def pack(a):
        ab = a.astype(jnp.bfloat16)
        pair = jnp.stack([ab[:, :HALF], ab[:, HALF:]], axis=-1)
        return lax.bitcast_convert_type(pair, jnp.float32)

    A = _tc_pre(xn, msg_W1[:RAW])
    table = jnp.concatenate([pack(x), A], axis=1)
    grow, gcol = _sc_gather(table, edge_index[0], edge_index[1])

    nq = qry_embeds.shape[0]
    nq_pad = ((nq + 1 + NS * C - 1) // (NS * C)) * (NS * C)
    B = 4000
    row3 = edge_index[0].reshape(E // B, 1, B)
    bf = jnp.bfloat16
    w1b = msg_W1[RAW:RAW + 1]
    trans, rowq3 = _tc_mlp(grow, gcol, row3, n_proto, nq_pad - 1,
                           w1b, msg_b1.reshape(1, -1),
                           msg_W2.astype(bf), msg_b2.reshape(1, -1),
                           trans_W1.astype(bf), trans_b1.reshape(1, -1),
                           trans_W2.astype(bf), trans_b2.reshape(1, -1),
                           trans_W3.reshape(1, -1))

    zrow = jnp.zeros((C, RAW), jnp.float32)
    ones_c = jnp.ones((C, RAW), jnp.float32)
    s_parts, cnt_parts = _sc_scatter(trans, rowq3.reshape(E), zrow, ones_c,
                                     nq_pad)

    qry_new = _tc_finish(qry_embeds, s_parts, cnt_parts)
    return (neighbor, qry_new)


# speedup vs baseline: 1.6402x; 1.1552x over previous
import functools

import jax
import jax.numpy as jnp
from jax import lax
from jax.experimental import pallas as pl
from jax.experimental.pallas import tpu as pltpu
from jax.experimental.pallas import tpu_sc as plsc

NC = 2
NS = 16
NW = NC * NS
C = 40
RAW = 128


HALF = RAW // 2


def _sc_gather(table, row, col):
    E = row.shape[0]
    per_w = E // NW
    nch = per_w // C
    n_pairs = nch // 2
    mesh = plsc.VectorSubcoreMesh(
        core_axis_name="c", subcore_axis_name="s", num_cores=NC,
        num_subcores=NS)
    fdt = jax.ShapeDtypeStruct((E, RAW), jnp.float32)
    row_rs = row.reshape(NW, nch, C)
    col_rs = col.reshape(NW, nch, C)

    @functools.partial(
        pl.kernel, mesh=mesh,
        out_type=(fdt, fdt),
        scratch_types=[
            pltpu.VMEM((nch, C), jnp.int32),
            pltpu.VMEM((nch, C), jnp.int32),
            pltpu.VMEM((C, RAW), jnp.float32),
            pltpu.VMEM((C, RAW), jnp.float32),
            pltpu.VMEM((C, RAW), jnp.float32),
            pltpu.VMEM((C, RAW), jnp.float32),
            pltpu.SemaphoreType.DMA,
            pltpu.SemaphoreType.DMA,
        ],
    )
    def k(t_hbm, row_hbm, col_hbm, gr_out, gc_out,
          ri, ci, gr_a, gc_a, gr_b, gc_b, sem_a, sem_b):
        wid = lax.axis_index("s") * NC + lax.axis_index("c")
        w_base = wid * per_w
        pltpu.sync_copy(row_hbm.at[wid], ri)
        pltpu.sync_copy(col_hbm.at[wid], ci)

        def drain_b():
            pltpu.make_async_copy(t_hbm.at[pl.ds(0, C)], gr_b, sem_b).wait()
            pltpu.make_async_copy(t_hbm.at[pl.ds(0, C)], gc_b, sem_b).wait()

        def body(i, _):
            a = w_base + (2 * i) * C
            b = a + C
            da0 = pltpu.async_copy(t_hbm.at[ri.at[2 * i]], gr_a, sem_a)
            da1 = pltpu.async_copy(t_hbm.at[ci.at[2 * i]], gc_a, sem_a)

            @pl.when(i > 0)
            def _prev():
                bp = a - C
                drain_b()
                pltpu.sync_copy(gr_b, gr_out.at[pl.ds(bp, C)])
                pltpu.sync_copy(gc_b, gc_out.at[pl.ds(bp, C)])

            pltpu.async_copy(t_hbm.at[ri.at[2 * i + 1]], gr_b, sem_b)
            pltpu.async_copy(t_hbm.at[ci.at[2 * i + 1]], gc_b, sem_b)
            da0.wait()
            da1.wait()
            pltpu.sync_copy(gr_a, gr_out.at[pl.ds(a, C)])
            pltpu.sync_copy(gc_a, gc_out.at[pl.ds(a, C)])
            return _

        lax.fori_loop(0, n_pairs, body, None)
        bl = w_base + per_w - C
        drain_b()
        pltpu.sync_copy(gr_b, gr_out.at[pl.ds(bl, C)])
        pltpu.sync_copy(gc_b, gc_out.at[pl.ds(bl, C)])

    return k(table, row_rs, col_rs)


def _tc_pre(xn, w1a):
    n = xn.shape[0]
    BP = 2000

    def body(xn_ref, w_ref, out_ref):
        out_ref[...] = jnp.dot(xn_ref[...], w_ref[...],
                               preferred_element_type=jnp.float32)

    return pl.pallas_call(
        body,
        grid=(n // BP,),
        in_specs=[pl.BlockSpec((BP, RAW), lambda i: (i, 0)),
                  pl.BlockSpec((RAW, HALF), lambda i: (0, 0))],
        out_specs=pl.BlockSpec((BP, HALF), lambda i: (i, 0)),
        out_shape=jax.ShapeDtypeStruct((n, HALF), jnp.float32),
    )(xn, w1a)


def _tc_mlp(grow, gcol, row3, n_proto, trash,
            w1b, b1, w2, b2, tw1, tb1, tw2, tb2, tw3):
    E = grow.shape[0]
    B = 4000
    grid = (E // B,)

    def unpack(packed):
        u = lax.bitcast_convert_type(packed, jnp.int32)
        lo = lax.bitcast_convert_type(u << 16, jnp.float32)
        hi = lax.bitcast_convert_type(u & jnp.int32(-65536), jnp.float32)
        return jnp.concatenate([lo, hi], axis=1)

    def body(gr_ref, gc_ref, row_ref, w1b_ref, b1_ref,
             w2_ref, b2_ref, tw1_ref, tb1_ref, tw2_ref, tb2_ref, tw3_ref,
             out_ref, rowq_ref):
        r = row_ref[0, 0, :]
        rowq_ref[0, 0, :] = jnp.where(r >= n_proto, r - n_proto, trash)
        diff = unpack(gr_ref[:, :HALF]) - unpack(gc_ref[:, :HALF])
        sqd = jnp.sum(diff * diff, axis=1, keepdims=True)
        h = gc_ref[:, HALF:] + sqd * w1b_ref[...] + b1_ref[...]
        h = h * jax.nn.sigmoid(h)
        h = jnp.dot(h.astype(jnp.bfloat16), w2_ref[...],
                    preferred_element_type=jnp.float32)
        h = h + b2_ref[...]
        h = h * jax.nn.sigmoid(h)
        h = jnp.dot(h.astype(jnp.bfloat16), tw1_ref[...],
                    preferred_element_type=jnp.float32)
        h = h + tb1_ref[...]
        h = h * jax.nn.sigmoid(h)
        h = jnp.dot(h.astype(jnp.bfloat16), tw2_ref[...],
                    preferred_element_type=jnp.float32)
        h = h + tb2_ref[...]
        h = h * jax.nn.sigmoid(h)
        t = jnp.sum(h * tw3_ref[...], axis=1, keepdims=True)
        out_ref[...] = diff * t

    blk_o = pl.BlockSpec((B, RAW), lambda i: (i, 0))
    blk_r = pl.BlockSpec((1, 1, B), lambda i: (i, 0, 0))
    full = lambda shape: pl.BlockSpec(shape, lambda i: tuple(0 for _ in shape))
    return pl.pallas_call(
        body,
        grid=grid,
        in_specs=[
            blk_o, blk_o, blk_r,
            full((1, 64)), full((1, 64)),
            full((64, 64)), full((1, 64)),
            full((64, 64)), full((1, 64)),
            full((64, 64)), full((1, 64)),
            full((1, 64)),
        ],
        out_specs=(blk_o, blk_r),
        out_shape=(jax.ShapeDtypeStruct((E, RAW), jnp.float32),
                   jax.ShapeDtypeStruct((E // B, 1, B), jnp.int32)),
    )(grow, gcol, row3, w1b, b1, w2, b2, tw1, tb1, tw2, tb2, tw3)


def _sc_scatter(trans, rowq, zrow, ones_c, NQ):
    E, _ = trans.shape
    per_w = E // NW
    n_chunks = per_w // C
    rows_per_tile = NQ // NS
    n_init = rows_per_tile // C
    mesh = plsc.VectorSubcoreMesh(
        core_axis_name="c", subcore_axis_name="s", num_cores=NC,
        num_subcores=NS)
    fdt = jax.ShapeDtypeStruct((NC, NQ, RAW), jnp.float32)
    rowq_rs = rowq.reshape(NW, n_chunks, C)

    n_pairs = n_chunks // 2

    @functools.partial(
        pl.kernel, mesh=mesh,
        out_type=(fdt, fdt),
        scratch_types=[
            pltpu.VMEM_SHARED((NQ, RAW), jnp.float32),
            pltpu.VMEM_SHARED((NQ, RAW), jnp.float32),
            pltpu.VMEM((n_chunks, C), jnp.int32),
            pltpu.VMEM((C, RAW), jnp.float32),
            pltpu.VMEM((C, RAW), jnp.float32),
            pltpu.VMEM((C, RAW), jnp.float32),
            pltpu.SemaphoreType.DMA,
            pltpu.SemaphoreType.DMA,
        ],
    )
    def k(trans_hbm, rowq_hbm, zrow_hbm, ones_hbm, s_out, cnt_out,
          acc_sh, cnt_sh, rq, tr_a, tr_b, ones_v, sem_a, sem_b):
        cid = lax.axis_index("c")
        sid = lax.axis_index("s")
        wid = sid * NC + cid
        w_base = wid * per_w
        tile_rows = sid * rows_per_tile
        pltpu.sync_copy(rowq_hbm.at[wid], rq)

        pltpu.sync_copy(zrow_hbm, tr_a)

        def zbody(j, _):
            pltpu.sync_copy(tr_a, acc_sh.at[pl.ds(tile_rows + j * C, C)])
            pltpu.sync_copy(tr_a, cnt_sh.at[pl.ds(tile_rows + j * C, C)])
            return _

        lax.fori_loop(0, n_init, zbody, None)
        pltpu.sync_copy(ones_hbm, ones_v)
        plsc.subcore_barrier()

        def scat_b(j):
            pltpu.make_async_copy(
                trans_hbm.at[pl.ds(0, C)], tr_b, sem_b).wait()
            pltpu.sync_copy(tr_b, acc_sh.at[rq.at[j]], add=True)
            pltpu.sync_copy(ones_v, cnt_sh.at[rq.at[j]], add=True)

        def body(i, _):
            a = w_base + (2 * i) * C
            b = a + C
            da = pltpu.async_copy(trans_hbm.at[pl.ds(a, C)], tr_a, sem_a)

            @pl.when(i > 0)
            def _prev():
                scat_b(2 * i - 1)

            pltpu.async_copy(trans_hbm.at[pl.ds(b, C)], tr_b, sem_b)
            da.wait()
            pltpu.sync_copy(tr_a, acc_sh.at[rq.at[2 * i]], add=True)
            pltpu.sync_copy(ones_v, cnt_sh.at[rq.at[2 * i]], add=True)
            return _

        lax.fori_loop(0, n_pairs, body, None)
        scat_b(n_chunks - 1)
        plsc.subcore_barrier()

        def obody(j, _):
            r = tile_rows + j * C
            pltpu.sync_copy(acc_sh.at[pl.ds(r, C)], tr_a)
            pltpu.sync_copy(tr_a, s_out.at[cid, pl.ds(r, C)])
            pltpu.sync_copy(cnt_sh.at[pl.ds(r, C)], ones_v)
            pltpu.sync_copy(ones_v, cnt_out.at[cid, pl.ds(r, C)])
            return _

        lax.fori_loop(0, n_init, obody, None)

    return k(trans, rowq_rs, zrow, ones_c)


def _tc_finish(qry, s_parts, cnt_parts):
    nq = qry.shape[0]

    def body(qry_ref, s_ref, cnt_ref, out_ref):
        s = s_ref[0] + s_ref[1]
        cnt = cnt_ref[0, :, 0:1] + cnt_ref[1, :, 0:1]
        out_ref[...] = qry_ref[...] + s / jnp.maximum(cnt, 1.0)

    return pl.pallas_call(
        body,
        grid=(1,),
        in_specs=[
            pl.BlockSpec((nq, RAW), lambda i: (0, 0)),
            pl.BlockSpec((NC, nq, RAW), lambda i: (0, 0, 0)),
            pl.BlockSpec((NC, nq, RAW), lambda i: (0, 0, 0)),
        ],
        out_specs=pl.BlockSpec((nq, RAW), lambda i: (0, 0)),
        out_shape=jax.ShapeDtypeStruct((nq, RAW), jnp.float32),
    )(qry, s_parts, cnt_parts)


def kernel(edge_index, neighbor, qry_embeds, proto_embeds,
           msg_W1, msg_b1, msg_W2, msg_b2,
           trans_W1, trans_b1, trans_W2, trans_b2, trans_W3):
    n_proto = proto_embeds.shape[0]
    x = jnp.concatenate([proto_embeds, qry_embeds], axis=0)
    xn = jnp.concatenate([proto_embeds, neighbor], axis=0)
    E = edge_index.shape[1]

    def pack(a):
        ab = a.astype(jnp.bfloat16)
        pair = jnp.stack([ab[:, :HALF], ab[:, HALF:]], axis=-1)
        return lax.bitcast_convert_type(pair, jnp.float32)

    A = _tc_pre(xn, msg_W1[:RAW])
    table = jnp.concatenate([pack(x), A], axis=1)
    grow, gcol = _sc_gather(table, edge_index[0], edge_index[1])

    nq = qry_embeds.shape[0]
    nq_pad = ((nq + 1 + NS * C - 1) // (NS * C)) * (NS * C)
    B = 4000
    row3 = edge_index[0].reshape(E // B, 1, B)
    bf = jnp.bfloat16
    w1b = msg_W1[RAW:RAW + 1]
    trans, rowq3 = _tc_mlp(grow, gcol, row3, n_proto, nq_pad - 1,
                           w1b, msg_b1.reshape(1, -1),
                           msg_W2.astype(bf), msg_b2.reshape(1, -1),
                           trans_W1.astype(bf), trans_b1.reshape(1, -1),
                           trans_W2.astype(bf), trans_b2.reshape(1, -1),
                           trans_W3.reshape(1, -1))

    zrow = jnp.zeros((C, RAW), jnp.float32)
    ones_c = jnp.ones((C, RAW), jnp.float32)
    s_parts, cnt_parts = _sc_scatter(trans, rowq3.reshape(E), zrow, ones_c,
                                     nq_pad)

    qry_new = _tc_finish(qry_embeds, s_parts, cnt_parts)
    return (neighbor, qry_new)


# speedup vs baseline: 2.0290x; 1.2370x over previous
import functools

import jax
import jax.numpy as jnp
from jax import lax
from jax.experimental import pallas as pl
from jax.experimental.pallas import tpu as pltpu
from jax.experimental.pallas import tpu_sc as plsc

NC = 2
NS = 16
NW = NC * NS
C = 40
RAW = 128


HALF = RAW // 2


def _sc_gather(table, row, col):
    E = row.shape[0]
    per_w = E // NW
    nch = per_w // C
    n_pairs = nch // 2
    mesh = plsc.VectorSubcoreMesh(
        core_axis_name="c", subcore_axis_name="s", num_cores=NC,
        num_subcores=NS)
    fdt = jax.ShapeDtypeStruct((E, RAW), jnp.float32)
    row_rs = row.reshape(NW, nch, C)
    col_rs = col.reshape(NW, nch, C)

    @functools.partial(
        pl.kernel, mesh=mesh,
        out_type=(fdt, fdt),
        scratch_types=[
            pltpu.VMEM((nch, C), jnp.int32),
            pltpu.VMEM((nch, C), jnp.int32),
            pltpu.VMEM((C, RAW), jnp.float32),
            pltpu.VMEM((C, RAW), jnp.float32),
            pltpu.VMEM((C, RAW), jnp.float32),
            pltpu.VMEM((C, RAW), jnp.float32),
            pltpu.SemaphoreType.DMA,
            pltpu.SemaphoreType.DMA,
        ],
    )
    def k(t_hbm, row_hbm, col_hbm, gr_out, gc_out,
          ri, ci, gr_a, gc_a, gr_b, gc_b, sem_a, sem_b):
        wid = lax.axis_index("s") * NC + lax.axis_index("c")
        w_base = wid * per_w
        pltpu.sync_copy(row_hbm.at[wid], ri)
        pltpu.sync_copy(col_hbm.at[wid], ci)

        def drain_b():
            pltpu.make_async_copy(t_hbm.at[pl.ds(0, C)], gr_b, sem_b).wait()
            pltpu.make_async_copy(t_hbm.at[pl.ds(0, C)], gc_b, sem_b).wait()

        def body(i, _):
            a = w_base + (2 * i) * C
            b = a + C
            da0 = pltpu.async_copy(t_hbm.at[ri.at[2 * i]], gr_a, sem_a)
            da1 = pltpu.async_copy(t_hbm.at[ci.at[2 * i]], gc_a, sem_a)

            @pl.when(i > 0)
            def _prev():
                bp = a - C
                drain_b()
                pltpu.sync_copy(gr_b, gr_out.at[pl.ds(bp, C)])
                pltpu.sync_copy(gc_b, gc_out.at[pl.ds(bp, C)])

            pltpu.async_copy(t_hbm.at[ri.at[2 * i + 1]], gr_b, sem_b)
            pltpu.async_copy(t_hbm.at[ci.at[2 * i + 1]], gc_b, sem_b)
            da0.wait()
            da1.wait()
            pltpu.sync_copy(gr_a, gr_out.at[pl.ds(a, C)])
            pltpu.sync_copy(gc_a, gc_out.at[pl.ds(a, C)])
            return _

        lax.fori_loop(0, n_pairs, body, None)
        if nch % 2 == 0:
            bl = w_base + per_w - C
            drain_b()
            pltpu.sync_copy(gr_b, gr_out.at[pl.ds(bl, C)])
            pltpu.sync_copy(gc_b, gc_out.at[pl.ds(bl, C)])
        else:
            bl = w_base + per_w - C
            da0 = pltpu.async_copy(t_hbm.at[ri.at[nch - 1]], gr_a, sem_a)
            da1 = pltpu.async_copy(t_hbm.at[ci.at[nch - 1]], gc_a, sem_a)
            drain_b()
            pltpu.sync_copy(gr_b, gr_out.at[pl.ds(bl - C, C)])
            pltpu.sync_copy(gc_b, gc_out.at[pl.ds(bl - C, C)])
            da0.wait()
            da1.wait()
            pltpu.sync_copy(gr_a, gr_out.at[pl.ds(bl, C)])
            pltpu.sync_copy(gc_a, gc_out.at[pl.ds(bl, C)])

    return k(table, row_rs, col_rs)


def _tc_pre(xn, w1a):
    n = xn.shape[0]
    BP = 2000

    def body(xn_ref, w_ref, out_ref):
        out_ref[...] = jnp.dot(xn_ref[...], w_ref[...],
                               preferred_element_type=jnp.float32)

    return pl.pallas_call(
        body,
        grid=(n // BP,),
        in_specs=[pl.BlockSpec((BP, RAW), lambda i: (i, 0)),
                  pl.BlockSpec((RAW, HALF), lambda i: (0, 0))],
        out_specs=pl.BlockSpec((BP, HALF), lambda i: (i, 0)),
        out_shape=jax.ShapeDtypeStruct((n, HALF), jnp.float32),
    )(xn, w1a)


def _tc_mlp(grow, gcol, row3, n_proto, trash,
            w1b, b1, w2, b2, tw1, tb1, tw2, tb2, tw3):
    E = grow.shape[0]
    B = 4000
    grid = (E // B,)

    def unpack(packed):
        u = lax.bitcast_convert_type(packed, jnp.int32)
        lo = lax.bitcast_convert_type(u << 16, jnp.float32)
        hi = lax.bitcast_convert_type(u & jnp.int32(-65536), jnp.float32)
        return jnp.concatenate([lo, hi], axis=1)

    def body(gr_ref, gc_ref, row_ref, w1b_ref, b1_ref,
             w2_ref, b2_ref, tw1_ref, tb1_ref, tw2_ref, tb2_ref, tw3_ref,
             out_ref, rowq_ref):
        r = row_ref[0, 0, :]
        rowq_ref[0, 0, :] = jnp.where(r >= n_proto, r - n_proto, trash)
        diff = unpack(gr_ref[:, :HALF]) - unpack(gc_ref[:, :HALF])
        sqd = jnp.sum(diff * diff, axis=1, keepdims=True)
        h = gc_ref[:, HALF:] + sqd * w1b_ref[...] + b1_ref[...]
        h = h * jax.nn.sigmoid(h)
        h = jnp.dot(h.astype(jnp.bfloat16), w2_ref[...],
                    preferred_element_type=jnp.float32)
        h = h + b2_ref[...]
        h = h * jax.nn.sigmoid(h)
        h = jnp.dot(h.astype(jnp.bfloat16), tw1_ref[...],
                    preferred_element_type=jnp.float32)
        h = h + tb1_ref[...]
        h = h * jax.nn.sigmoid(h)
        h = jnp.dot(h.astype(jnp.bfloat16), tw2_ref[...],
                    preferred_element_type=jnp.float32)
        h = h + tb2_ref[...]
        h = h * jax.nn.sigmoid(h)
        t = jnp.sum(h * tw3_ref[...], axis=1, keepdims=True)
        out_ref[...] = diff * t

    blk_o = pl.BlockSpec((B, RAW), lambda i: (i, 0))
    blk_r = pl.BlockSpec((1, 1, B), lambda i: (i, 0, 0))
    full = lambda shape: pl.BlockSpec(shape, lambda i: tuple(0 for _ in shape))
    return pl.pallas_call(
        body,
        grid=grid,
        in_specs=[
            blk_o, blk_o, blk_r,
            full((1, 64)), full((1, 64)),
            full((64, 64)), full((1, 64)),
            full((64, 64)), full((1, 64)),
            full((64, 64)), full((1, 64)),
            full((1, 64)),
        ],
        out_specs=(blk_o, blk_r),
        out_shape=(jax.ShapeDtypeStruct((E, RAW), jnp.float32),
                   jax.ShapeDtypeStruct((E // B, 1, B), jnp.int32)),
    )(grow, gcol, row3, w1b, b1, w2, b2, tw1, tb1, tw2, tb2, tw3)


def _sc_scatter(trans, rowq, zrow, ones_c, NQ):
    E, _ = trans.shape
    per_w = E // NW
    n_chunks = per_w // C
    rows_per_tile = NQ // NS
    n_init = rows_per_tile // C
    mesh = plsc.VectorSubcoreMesh(
        core_axis_name="c", subcore_axis_name="s", num_cores=NC,
        num_subcores=NS)
    fdt = jax.ShapeDtypeStruct((NC, NQ, RAW), jnp.float32)
    rowq_rs = rowq.reshape(NW, n_chunks, C)

    n_pairs = n_chunks // 2

    @functools.partial(
        pl.kernel, mesh=mesh,
        out_type=(fdt, fdt),
        scratch_types=[
            pltpu.VMEM_SHARED((NQ, RAW), jnp.float32),
            pltpu.VMEM_SHARED((NQ, RAW), jnp.float32),
            pltpu.VMEM((n_chunks, C), jnp.int32),
            pltpu.VMEM((C, RAW), jnp.float32),
            pltpu.VMEM((C, RAW), jnp.float32),
            pltpu.VMEM((C, RAW), jnp.float32),
            pltpu.SemaphoreType.DMA,
            pltpu.SemaphoreType.DMA,
        ],
    )
    def k(trans_hbm, rowq_hbm, zrow_hbm, ones_hbm, s_out, cnt_out,
          acc_sh, cnt_sh, rq, tr_a, tr_b, ones_v, sem_a, sem_b):
        cid = lax.axis_index("c")
        sid = lax.axis_index("s")
        wid = sid * NC + cid
        w_base = wid * per_w
        tile_rows = sid * rows_per_tile
        pltpu.sync_copy(rowq_hbm.at[wid], rq)

        pltpu.sync_copy(zrow_hbm, tr_a)

        def zbody(j, _):
            pltpu.sync_copy(tr_a, acc_sh.at[pl.ds(tile_rows + j * C, C)])
            pltpu.sync_copy(tr_a, cnt_sh.at[pl.ds(tile_rows + j * C, C)])
            return _

        lax.fori_loop(0, n_init, zbody, None)
        pltpu.sync_copy(ones_hbm, ones_v)
        plsc.subcore_barrier()

        def scat_b(j):
            pltpu.make_async_copy(
                trans_hbm.at[pl.ds(0, C)], tr_b, sem_b).wait()
            pltpu.sync_copy(tr_b, acc_sh.at[rq.at[j]], add=True)
            pltpu.sync_copy(ones_v, cnt_sh.at[rq.at[j]], add=True)

        def body(i, _):
            a = w_base + (2 * i) * C
            b = a + C
            da = pltpu.async_copy(trans_hbm.at[pl.ds(a, C)], tr_a, sem_a)

            @pl.when(i > 0)
            def _prev():
                scat_b(2 * i - 1)

            pltpu.async_copy(trans_hbm.at[pl.ds(b, C)], tr_b, sem_b)
            da.wait()
            pltpu.sync_copy(tr_a, acc_sh.at[rq.at[2 * i]], add=True)
            pltpu.sync_copy(ones_v, cnt_sh.at[rq.at[2 * i]], add=True)
            return _

        lax.fori_loop(0, n_pairs, body, None)
        if n_chunks % 2 == 0:
            scat_b(n_chunks - 1)
        else:
            da = pltpu.async_copy(
                trans_hbm.at[pl.ds(w_base + per_w - C, C)], tr_a, sem_a)
            scat_b(n_chunks - 2)
            da.wait()
            pltpu.sync_copy(tr_a, acc_sh.at[rq.at[n_chunks - 1]], add=True)
            pltpu.sync_copy(ones_v, cnt_sh.at[rq.at[n_chunks - 1]], add=True)
        plsc.subcore_barrier()

        def obody(j, _):
            r = tile_rows + j * C
            pltpu.sync_copy(acc_sh.at[pl.ds(r, C)], tr_a)
            pltpu.sync_copy(tr_a, s_out.at[cid, pl.ds(r, C)])
            pltpu.sync_copy(cnt_sh.at[pl.ds(r, C)], ones_v)
            pltpu.sync_copy(ones_v, cnt_out.at[cid, pl.ds(r, C)])
            return _

        lax.fori_loop(0, n_init, obody, None)

    return k(trans, rowq_rs, zrow, ones_c)


def _tc_finish(qry, s_parts, cnt_parts):
    nq = qry.shape[0]
    P = s_parts.shape[0]

    def body(qry_ref, s_ref, cnt_ref, out_ref):
        s = s_ref[0]
        cnt = cnt_ref[0, :, 0:1]
        for p in range(1, P):
            s = s + s_ref[p]
            cnt = cnt + cnt_ref[p, :, 0:1]
        out_ref[...] = qry_ref[...] + s / jnp.maximum(cnt, 1.0)

    return pl.pallas_call(
        body,
        grid=(1,),
        in_specs=[
            pl.BlockSpec((nq, RAW), lambda i: (0, 0)),
            pl.BlockSpec((P, nq, RAW), lambda i: (0, 0, 0)),
            pl.BlockSpec((P, nq, RAW), lambda i: (0, 0, 0)),
        ],
        out_specs=pl.BlockSpec((nq, RAW), lambda i: (0, 0)),
        out_shape=jax.ShapeDtypeStruct((nq, RAW), jnp.float32),
    )(qry, s_parts, cnt_parts)


def kernel(edge_index, neighbor, qry_embeds, proto_embeds,
           msg_W1, msg_b1, msg_W2, msg_b2,
           trans_W1, trans_b1, trans_W2, trans_b2, trans_W3):
    n_proto = proto_embeds.shape[0]
    x = jnp.concatenate([proto_embeds, qry_embeds], axis=0)
    xn = jnp.concatenate([proto_embeds, neighbor], axis=0)
    E = edge_index.shape[1]

    def pack(a):
        ab = a.astype(jnp.bfloat16)
        pair = jnp.stack([ab[:, :HALF], ab[:, HALF:]], axis=-1)
        return lax.bitcast_convert_type(pair, jnp.float32)

    A = _tc_pre(xn, msg_W1[:RAW])
    table = jnp.concatenate([pack(x), A], axis=1)

    nq = qry_embeds.shape[0]
    nq_pad = ((nq + 1 + NS * C - 1) // (NS * C)) * (NS * C)
    B = 4000
    bf = jnp.bfloat16
    w1b = msg_W1[RAW:RAW + 1]
    zrow = jnp.zeros((C, RAW), jnp.float32)
    ones_c = jnp.ones((C, RAW), jnp.float32)

    h = E // 2
    parts = []
    for lohi in range(2):
        sl = slice(lohi * h, (lohi + 1) * h)
        row_h = edge_index[0, sl]
        grow, gcol = _sc_gather(table, row_h, edge_index[1, sl])
        row3 = row_h.reshape(h // B, 1, B)
        trans, rowq3 = _tc_mlp(grow, gcol, row3, n_proto, nq_pad - 1,
                               w1b, msg_b1.reshape(1, -1),
                               msg_W2.astype(bf), msg_b2.reshape(1, -1),
                               trans_W1.astype(bf), trans_b1.reshape(1, -1),
                               trans_W2.astype(bf), trans_b2.reshape(1, -1),
                               trans_W3.reshape(1, -1))
        parts.append(_sc_scatter(trans, rowq3.reshape(h), zrow, ones_c,
                                 nq_pad))

    s_parts = jnp.concatenate([parts[0][0], parts[1][0]], axis=0)
    cnt_parts = jnp.concatenate([parts[0][1], parts[1][1]], axis=0)
    qry_new = _tc_finish(qry_embeds, s_parts, cnt_parts)
    return (neighbor, qry_new)
